# baseline (speedup 1.0000x reference)
import functools

import jax
import jax.numpy as jnp
from jax import lax
from jax.experimental import pallas as pl
from jax.experimental.pallas import tpu as pltpu
from jax.experimental.pallas import tpu_sc as plsc

NW = 32
LANES = 16
RS = 33
WS = 1025


def _basew_body(x_ref, wm1_ref, bm1_ref, base_ref, w_ref):
    n = x_ref.shape[0]
    acc = bm1_ref[...]
    for k in range(9):
        acc = acc + x_ref[:, k:k + 1] * wm1_ref[:, k, :]
    zcol = jnp.zeros((n, 1), jnp.float32)
    base_ref[...] = jnp.concatenate([acc, zcol], axis=1)
    w_ref[...] = jnp.concatenate([wm1_ref[:, 9, :], zcol], axis=1)


def _make_edge_sc(n_nodes, e_pad, chunk):
    n_groups = chunk // LANES
    mesh = plsc.VectorSubcoreMesh(core_axis_name="c", subcore_axis_name="s")

    @functools.partial(
        pl.kernel,
        mesh=mesh,
        compiler_params=pltpu.CompilerParams(needs_layout_passes=False,
                                             use_tc_tiling_on_sc=False),
        out_type=jax.ShapeDtypeStruct((e_pad, RS), jnp.float32),
        scratch_types=[
            pltpu.VMEM((n_nodes, RS), jnp.float32),
            pltpu.VMEM((n_nodes, RS), jnp.float32),
            pltpu.VMEM((n_nodes, WS), jnp.float32),
            pltpu.VMEM((n_nodes, RS), jnp.float32),
            pltpu.VMEM((chunk,), jnp.int32),
            pltpu.VMEM((chunk,), jnp.float32),
            pltpu.VMEM((32 * LANES,), jnp.float32),
            pltpu.VMEM((chunk, RS), jnp.float32),
        ],
    )
    def edge_kernel(base_hbm, w_hbm, wm2_hbm, bm2_hbm, src_hbm, attr_hbm,
                    out_hbm, base_v, w_v, wm2_v, bm2_v, src_v, attr_v,
                    h1_v, out_v):
        wid = lax.axis_index("s") * 2 + lax.axis_index("c")
        pltpu.sync_copy(base_hbm, base_v)
        pltpu.sync_copy(w_hbm, w_v)
        pltpu.sync_copy(wm2_hbm, wm2_v)
        pltpu.sync_copy(bm2_hbm, bm2_v)
        pltpu.sync_copy(src_hbm.at[pl.ds(wid * chunk, chunk)], src_v)
        pltpu.sync_copy(attr_hbm.at[pl.ds(wid * chunk, chunk)], attr_v)

        lanes = lax.broadcasted_iota(jnp.int32, (LANES,), 0)

        def col(c):
            return jnp.full((LANES,), c, jnp.int32)

        def group(g, carry):
            sv = src_v[pl.ds(g * LANES, LANES)]
            av = attr_v[pl.ds(g * LANES, LANES)]
            s_safe = jnp.maximum(sv, 0)
            erow = g * LANES + lanes
            for k in range(32):
                h1k = plsc.load_gather(base_v, [s_safe, col(k)]) \
                    + av * plsc.load_gather(w_v, [s_safe, col(k)])
                h1_v[pl.ds(k * LANES, LANES)] = jnp.maximum(h1k, 0.0)

            for half in range(2):
                hb = half * LANES
                accs = [plsc.load_gather(bm2_v, [s_safe, col(hb + o)])
                        for o in range(LANES)]
                for k in range(32):
                    h1k = h1_v[pl.ds(k * LANES, LANES)]
                    cb = k * 32 + hb
                    for o in range(LANES):
                        wv = plsc.load_gather(wm2_v, [s_safe, col(cb + o)])
                        accs[o] = accs[o] + h1k * wv
                for o in range(LANES):
                    plsc.store_scatter(out_v, [erow, col(hb + o)],
                                       jnp.maximum(accs[o], 0.0))
            plsc.store_scatter(out_v, [erow, col(32)],
                               jnp.ones((LANES,), jnp.float32))
            return carry

        lax.fori_loop(0, n_groups, group, 0)
        pltpu.sync_copy(out_v, out_hbm.at[pl.ds(wid * chunk, chunk)])

    return edge_kernel


def _make_post(n_nodes, e_pad, tile):
    n_steps = e_pad // tile

    def post_body(src_ref, h2_ref, x_ref, bm1_ref, wm2_ref, bm2_ref,
                  wu1_ref, bu1_ref, wu2_ref, bu2_ref, wh1_ref, bh1_ref,
                  wh2_ref, bh2_ref, out_ref, acc_ref):
        i = pl.program_id(0)
        s = src_ref[...]
        cols = lax.broadcasted_iota(jnp.int32, (tile, n_nodes), 1)
        p = (s == cols).astype(jnp.float32)
        dn = (((0,), (0,)), ((), ()))
        partial = lax.dot_general(p, h2_ref[...], dn,
                                  preferred_element_type=jnp.float32)

        @pl.when(i == 0)
        def _():
            acc_ref[...] = partial

        @pl.when(i > 0)
        def _():
            acc_ref[...] += partial

        @pl.when(i == n_steps - 1)
        def _():
            aggr_sum = acc_ref[:, :32]
            deg = acc_ref[:, 32:33]
            z1 = jnp.maximum(bm1_ref[...], 0.0)
            acc = bm2_ref[...]
            for k in range(32):
                acc = acc + z1[:, k:k + 1] * wm2_ref[:, k, :]
            z2 = jnp.maximum(acc, 0.0)
            aggr = jnp.where(deg > 0.0, aggr_sum, z2)

            u = bu1_ref[...]
            for k in range(41):
                tk = x_ref[:, k:k + 1] if k < 9 else aggr[:, k - 9:k - 8]
                u = u + tk * wu1_ref[:, k, :]
            u = jnp.maximum(u, 0.0)

            comb = bu2_ref[...]
            for k in range(16):
                comb = comb + u[:, k:k + 1] * wu2_ref[:, k, :]
            comb = jnp.maximum(comb, 0.0)

            ph = bh1_ref[...]
            for k in range(8):
                ph = ph + comb[:, k:k + 1] * wh1_ref[:, k, :]
            ph = jnp.maximum(ph, 0.0)

            o = bh2_ref[...]
            for k in range(16):
                o = o + ph[:, k:k + 1] * wh2_ref[:, k, :]
            out_ref[...] = jax.nn.sigmoid(o)

    return post_body, n_steps


def kernel(x, edge_index, edge_attr, Wm1, bm1, Wm2, bm2,
           Wu1, bu1, Wu2, bu2, Wh1, bh1, Wh2, bh2):
    n = x.shape[0]
    e = edge_index.shape[1]
    src = edge_index[0]

    chunk = -(-e // (NW * LANES)) * LANES
    e_pad = NW * chunk
    src_pad = jnp.concatenate(
        [src, jnp.full((e_pad - e,), -1, jnp.int32)])
    attr_pad = jnp.concatenate(
        [edge_attr[:, 0], jnp.zeros((e_pad - e,), jnp.float32)])

    base, w = pl.pallas_call(
        _basew_body,
        out_shape=[jax.ShapeDtypeStruct((n, RS), jnp.float32),
                   jax.ShapeDtypeStruct((n, RS), jnp.float32)],
    )(x, Wm1, bm1)

    wm2_padded = jnp.pad(Wm2.reshape(n, 1024), ((0, 0), (0, WS - 1024)))
    bm2_padded = jnp.pad(bm2, ((0, 0), (0, RS - 32)))

    edge_sc = _make_edge_sc(n, e_pad, chunk)
    h2 = edge_sc(base, w, wm2_padded, bm2_padded, src_pad, attr_pad)

    tile = 2048
    post_body, n_steps = _make_post(n, e_pad, tile)
    full = lambda shape: pl.BlockSpec(shape, lambda i: tuple(0 for _ in shape))
    out = pl.pallas_call(
        post_body,
        grid=(n_steps,),
        in_specs=[
            pl.BlockSpec((tile, 1), lambda i: (i, 0)),
            pl.BlockSpec((tile, RS), lambda i: (i, 0)),
            full((n, 9)), full((n, 32)), full((n, 32, 32)), full((n, 32)),
            full((n, 41, 16)), full((n, 16)), full((n, 16, 8)), full((n, 8)),
            full((n, 8, 16)), full((n, 16)), full((n, 16, 1)), full((n, 1)),
        ],
        out_specs=full((n, 1)),
        out_shape=jax.ShapeDtypeStruct((n, 1), jnp.float32),
        scratch_shapes=[pltpu.VMEM((n, RS), jnp.float32)],
    )(src_pad.reshape(e_pad, 1), h2, x, bm1, Wm2, bm2,
      Wu1, bu1, Wu2, bu2, Wh1, bh1, Wh2, bh2)
    return out


# speedup vs baseline: 1.0720x; 1.0720x over previous
import functools

import jax
import jax.numpy as jnp
from jax import lax
from jax.experimental import pallas as pl
from jax.experimental.pallas import tpu as pltpu
from jax.experimental.pallas import tpu_sc as plsc

NW = 32
LANES = 16
RS = 33
WS = 1025


def _basew_body(x_ref, wm1_ref, bm1_ref, base_ref, w_ref):
    n = x_ref.shape[0]
    acc = bm1_ref[...]
    for k in range(9):
        acc = acc + x_ref[:, k:k + 1] * wm1_ref[:, k, :]
    zcol = jnp.zeros((n, 1), jnp.float32)
    base_ref[...] = jnp.concatenate([acc, zcol], axis=1)
    w_ref[...] = jnp.concatenate([wm1_ref[:, 9, :], zcol], axis=1)


def _make_edge_sc(n_nodes, e_pad, chunk):
    n_groups = chunk // LANES
    mesh = plsc.VectorSubcoreMesh(core_axis_name="c", subcore_axis_name="s")

    @functools.partial(
        pl.kernel,
        mesh=mesh,
        compiler_params=pltpu.CompilerParams(needs_layout_passes=False),
        out_type=jax.ShapeDtypeStruct((e_pad * RS,), jnp.float32),
        scratch_types=[
            pltpu.VMEM((n_nodes * RS,), jnp.float32),
            pltpu.VMEM((n_nodes * RS,), jnp.float32),
            pltpu.VMEM((n_nodes * WS,), jnp.float32),
            pltpu.VMEM((n_nodes * RS,), jnp.float32),
            pltpu.VMEM((chunk,), jnp.int32),
            pltpu.VMEM((chunk,), jnp.float32),
            pltpu.VMEM((32 * LANES,), jnp.float32),
            pltpu.VMEM((chunk * RS,), jnp.float32),
        ],
    )
    def edge_kernel(base_hbm, w_hbm, wm2_hbm, bm2_hbm, src_hbm, attr_hbm,
                    out_hbm, base_v, w_v, wm2_v, bm2_v, src_v, attr_v,
                    h1_v, out_v):
        wid = lax.axis_index("s") * 2 + lax.axis_index("c")
        pltpu.sync_copy(base_hbm, base_v)
        pltpu.sync_copy(w_hbm, w_v)
        pltpu.sync_copy(wm2_hbm, wm2_v)
        pltpu.sync_copy(bm2_hbm, bm2_v)
        pltpu.sync_copy(src_hbm.at[pl.ds(wid * chunk, chunk)], src_v)
        pltpu.sync_copy(attr_hbm.at[pl.ds(wid * chunk, chunk)], attr_v)

        lanes = lax.broadcasted_iota(jnp.int32, (LANES,), 0)

        def group(g, carry):
            sv = src_v[pl.ds(g * LANES, LANES)]
            av = attr_v[pl.ds(g * LANES, LANES)]
            s_safe = jnp.maximum(sv, 0)
            svr = s_safe * RS
            svw = s_safe * WS
            er = (g * LANES + lanes) * RS
            for k in range(32):
                idx = svr + k
                h1k = plsc.load_gather(base_v, [idx]) \
                    + av * plsc.load_gather(w_v, [idx])
                h1_v[pl.ds(k * LANES, LANES)] = jnp.maximum(h1k, 0.0)

            for half in range(2):
                hb = half * LANES
                accs = [plsc.load_gather(bm2_v, [svr + (hb + o)])
                        for o in range(LANES)]
                for k in range(32):
                    h1k = h1_v[pl.ds(k * LANES, LANES)]
                    idxbase = svw + (k * 32 + hb)
                    for o in range(LANES):
                        wv = plsc.load_gather(wm2_v, [idxbase + o])
                        accs[o] = accs[o] + h1k * wv
                for o in range(LANES):
                    plsc.store_scatter(out_v, [er + hb + o],
                                       jnp.maximum(accs[o], 0.0))
            plsc.store_scatter(out_v, [er + 32],
                               jnp.ones((LANES,), jnp.float32))
            return carry

        lax.fori_loop(0, n_groups, group, 0)
        pltpu.sync_copy(out_v, out_hbm.at[pl.ds(wid * chunk * RS, chunk * RS)])

    return edge_kernel


def _make_post(n_nodes, e_pad, tile):
    n_steps = e_pad // tile

    def post_body(src_ref, h2_ref, x_ref, bm1_ref, wm2_ref, bm2_ref,
                  wu1_ref, bu1_ref, wu2_ref, bu2_ref, wh1_ref, bh1_ref,
                  wh2_ref, bh2_ref, out_ref, acc_ref):
        i = pl.program_id(0)
        s = src_ref[...]
        cols = lax.broadcasted_iota(jnp.int32, (tile, n_nodes), 1)
        p = (s == cols).astype(jnp.float32)
        dn = (((0,), (0,)), ((), ()))
        partial = lax.dot_general(p, h2_ref[...], dn,
                                  preferred_element_type=jnp.float32)

        @pl.when(i == 0)
        def _():
            acc_ref[...] = partial

        @pl.when(i > 0)
        def _():
            acc_ref[...] += partial

        @pl.when(i == n_steps - 1)
        def _():
            aggr_sum = acc_ref[:, :32]
            deg = acc_ref[:, 32:33]
            z1 = jnp.maximum(bm1_ref[...], 0.0)
            acc = bm2_ref[...]
            for k in range(32):
                acc = acc + z1[:, k:k + 1] * wm2_ref[:, k, :]
            z2 = jnp.maximum(acc, 0.0)
            aggr = jnp.where(deg > 0.0, aggr_sum, z2)

            u = bu1_ref[...]
            for k in range(41):
                tk = x_ref[:, k:k + 1] if k < 9 else aggr[:, k - 9:k - 8]
                u = u + tk * wu1_ref[:, k, :]
            u = jnp.maximum(u, 0.0)

            comb = bu2_ref[...]
            for k in range(16):
                comb = comb + u[:, k:k + 1] * wu2_ref[:, k, :]
            comb = jnp.maximum(comb, 0.0)

            ph = bh1_ref[...]
            for k in range(8):
                ph = ph + comb[:, k:k + 1] * wh1_ref[:, k, :]
            ph = jnp.maximum(ph, 0.0)

            o = bh2_ref[...]
            for k in range(16):
                o = o + ph[:, k:k + 1] * wh2_ref[:, k, :]
            out_ref[...] = jax.nn.sigmoid(o)

    return post_body, n_steps


def kernel(x, edge_index, edge_attr, Wm1, bm1, Wm2, bm2,
           Wu1, bu1, Wu2, bu2, Wh1, bh1, Wh2, bh2):
    n = x.shape[0]
    e = edge_index.shape[1]
    src = edge_index[0]

    chunk = -(-e // (NW * LANES)) * LANES
    e_pad = NW * chunk
    src_pad = jnp.concatenate(
        [src, jnp.full((e_pad - e,), -1, jnp.int32)])
    attr_pad = jnp.concatenate(
        [edge_attr[:, 0], jnp.zeros((e_pad - e,), jnp.float32)])

    base, w = pl.pallas_call(
        _basew_body,
        out_shape=[jax.ShapeDtypeStruct((n, RS), jnp.float32),
                   jax.ShapeDtypeStruct((n, RS), jnp.float32)],
    )(x, Wm1, bm1)

    wm2_padded = jnp.pad(Wm2.reshape(n, 1024), ((0, 0), (0, WS - 1024)))
    bm2_padded = jnp.pad(bm2, ((0, 0), (0, RS - 32)))

    edge_sc = _make_edge_sc(n, e_pad, chunk)
    h2_flat = edge_sc(base.reshape(-1), w.reshape(-1), wm2_padded.reshape(-1),
                      bm2_padded.reshape(-1), src_pad, attr_pad)
    h2 = h2_flat.reshape(e_pad, RS)

    tile = 2048
    post_body, n_steps = _make_post(n, e_pad, tile)
    full = lambda shape: pl.BlockSpec(shape, lambda i: tuple(0 for _ in shape))
    out = pl.pallas_call(
        post_body,
        grid=(n_steps,),
        in_specs=[
            pl.BlockSpec((tile, 1), lambda i: (i, 0)),
            pl.BlockSpec((tile, RS), lambda i: (i, 0)),
            full((n, 9)), full((n, 32)), full((n, 32, 32)), full((n, 32)),
            full((n, 41, 16)), full((n, 16)), full((n, 16, 8)), full((n, 8)),
            full((n, 8, 16)), full((n, 16)), full((n, 16, 1)), full((n, 1)),
        ],
        out_specs=full((n, 1)),
        out_shape=jax.ShapeDtypeStruct((n, 1), jnp.float32),
        scratch_shapes=[pltpu.VMEM((n, RS), jnp.float32)],
    )(src_pad.reshape(e_pad, 1), h2, x, bm1, Wm2, bm2,
      Wu1, bu1, Wu2, bu2, Wh1, bh1, Wh2, bh2)
    return out
